# baseline (speedup 1.0000x reference)
import functools

import jax
import jax.numpy as jnp
from jax import lax
from jax.experimental import pallas as pl
from jax.experimental.pallas import tpu as pltpu, tpu_sc as plsc

HIDDEN = 64
LANES = 16
NUM_CORES = 2
NUM_SUBCORES = 16
NW = NUM_CORES * NUM_SUBCORES

SEQ = 200
BATCH = 4096
ST = SEQ // 8
BT = BATCH // 128
SI_PER_UNIT = 4
UNIT_ROWS = SI_PER_UNIT * 128
N_UNITS = ST * BT * (8 // SI_PER_UNIT)
UPW = N_UNITS // NW
NBUF = 2


def _coords(ug):
    st = ug // 64
    rem = ug % 64
    j = rem // 2
    half = rem % 2
    return st, j, half


def _emb_body(seq_hbm, pid_hbm, tok_hbm, pos_hbm, out_hbm,
              idxb0, idxb1, pidxb0, pidxb1, trows0, trows1, hbuf, pos_local,
              isem0, isem1, tsem0, tsem1, osem, psem):
    idxb = [idxb0, idxb1]
    pidxb = [pidxb0, pidxb1]
    trows = [trows0, trows1]
    isem = [isem0, isem1]
    tsem = [tsem0, tsem1]

    wid = lax.axis_index("s") * NUM_CORES + lax.axis_index("c")
    u0 = wid * UPW
    u_end = u0 + UPW

    iota16 = lax.iota(jnp.int32, LANES)
    bvec = [[iota16 + (sp * 128 + k * LANES) for k in range(8)]
            for sp in range(SI_PER_UNIT)]

    def stage(b, ug):
        st, j, half = _coords(ug)
        sl = pl.ds(UNIT_ROWS * half, UNIT_ROWS)
        pltpu.async_copy(seq_hbm.at[st, j, sl], idxb[b], isem[b])
        pltpu.async_copy(pid_hbm.at[st, j, sl], pidxb[b], isem[b])

    def wait_idx(b):
        pltpu.make_async_copy(seq_hbm.at[0, 0, pl.ds(0, UNIT_ROWS)],
                              idxb[b], isem[b]).wait()
        pltpu.make_async_copy(pid_hbm.at[0, 0, pl.ds(0, UNIT_ROWS)],
                              pidxb[b], isem[b]).wait()

    def start_tok(b):
        pltpu.async_copy(tok_hbm.at[idxb[b]], trows[b], tsem[b])

    def wait_tok(b):
        pltpu.make_async_copy(tok_hbm.at[idxb[b]], trows[b], tsem[b]).wait()

    def wait_out():
        pltpu.make_async_copy(
            hbuf, out_hbm.at[pl.ds(0, SI_PER_UNIT), slice(None), 0],
            osem).wait()

    pltpu.async_copy(pos_hbm, pos_local, psem).wait()
    stage(0, u0)
    stage(1, u0 + 1)
    wait_idx(0)
    start_tok(0)

    def group_body(g, carry):
        for b in range(NBUF):
            u = u0 + g * NBUF + b
            nb = (b + 1) % NBUF

            wait_tok(b)

            @pl.when(u + 1 < u_end)
            def _():
                wait_idx(nb)
                start_tok(nb)

            @pl.when(u > u0)
            def _():
                wait_out()

            def ht_body(ht, c):
                for sp in range(SI_PER_UNIT):
                    for k in range(8):
                        pid_vec = pidxb[b][pl.ds(sp * 128 + k * LANES, LANES)]
                        for hi in range(8):
                            hvec = jnp.full((LANES,), ht * 8 + hi, jnp.int32)
                            vals = (plsc.load_gather(trows[b], [bvec[sp][k], hvec])
                                    + plsc.load_gather(pos_local, [pid_vec, hvec]))
                            hbuf[sp, ht, hi, pl.ds(k * LANES, LANES)] = vals
                return c

            lax.fori_loop(0, 8, ht_body, 0, unroll=False)

            st, j, half = _coords(u)
            pltpu.async_copy(
                hbuf,
                out_hbm.at[pl.ds(st * 8 + SI_PER_UNIT * half, SI_PER_UNIT),
                           slice(None), j],
                osem)

            @pl.when(u + 2 < u_end)
            def _():
                stage(b, u + 2)
        return carry

    lax.fori_loop(0, UPW // NBUF, group_body, 0, unroll=False)
    wait_out()


def kernel(sequence, position_ids, token_table, pos_table):
    seq_r = (sequence.T.reshape(ST, 8, BT, 128).transpose(0, 2, 1, 3)
             .reshape(ST, BT, 1024).astype(jnp.int32))
    pid_r = (position_ids.T.reshape(ST, 8, BT, 128).transpose(0, 2, 1, 3)
             .reshape(ST, BT, 1024).astype(jnp.int32))

    mesh = plsc.VectorSubcoreMesh(core_axis_name="c", subcore_axis_name="s",
                                  num_cores=NUM_CORES, num_subcores=NUM_SUBCORES)
    scratch = (
        [pltpu.VMEM((UNIT_ROWS,), jnp.int32) for _ in range(2 * NBUF)]
        + [pltpu.VMEM((UNIT_ROWS, HIDDEN), jnp.float32) for _ in range(NBUF)]
        + [pltpu.VMEM((SI_PER_UNIT, 8, 8, 128), jnp.float32)]
        + [pltpu.VMEM((SEQ, HIDDEN), jnp.float32)]
        + [pltpu.SemaphoreType.DMA for _ in range(2 * NBUF + 2)]
    )
    emb = functools.partial(
        pl.kernel,
        out_type=jax.ShapeDtypeStruct((SEQ, 8, BT, 8, 128), jnp.float32),
        mesh=mesh,
        scratch_types=scratch,
        compiler_params=pltpu.CompilerParams(use_tc_tiling_on_sc=False,
                                             needs_layout_passes=False),
    )(_emb_body)

    x = emb(seq_r, pid_r, token_table, pos_table)
    return x.transpose(2, 4, 0, 1, 3).reshape(BATCH, SEQ, HIDDEN)


# speedup vs baseline: 1.7312x; 1.7312x over previous
import functools

import jax
import jax.numpy as jnp
from jax import lax
from jax.experimental import pallas as pl
from jax.experimental.pallas import tpu as pltpu, tpu_sc as plsc

HIDDEN = 64
LANES = 16
NUM_CORES = 2
NUM_SUBCORES = 16
NW = NUM_CORES * NUM_SUBCORES

SEQ = 200
BATCH = 4096
ST = SEQ // 8
BT = BATCH // 128
SI_PER_UNIT = 4
UNIT_ROWS = SI_PER_UNIT * 128
N_UNITS = ST * BT * (8 // SI_PER_UNIT)
UPW = N_UNITS // NW
NBUF = 2


def _coords(ug):
    st = ug // 64
    rem = ug % 64
    j = rem // 2
    half = rem % 2
    return st, j, half


def _emb_body(seq_hbm, pid_hbm, tok_hbm, pos_hbm, out_hbm,
              idxb0, idxb1, pidxb0, pidxb1, trows0, trows1, hbuf,
              isem0, isem1, psem0, psem1, tsem0, tsem1, osem):
    idxb = [idxb0, idxb1]
    pidxb = [pidxb0, pidxb1]
    trows = [trows0, trows1]
    isem = [isem0, isem1]
    psem = [psem0, psem1]
    tsem = [tsem0, tsem1]

    wid = lax.axis_index("s") * NUM_CORES + lax.axis_index("c")
    u0 = wid * UPW
    u_end = u0 + UPW

    iota16 = lax.iota(jnp.int32, LANES)
    hb_const = [(iota16 + 16 * hb) * 128 for hb in range(HIDDEN // LANES)]

    def stage(b, ug):
        st, j, half = _coords(ug)
        sl = pl.ds(UNIT_ROWS * half, UNIT_ROWS)
        pltpu.async_copy(seq_hbm.at[st, j, sl], idxb[b], isem[b])
        pltpu.async_copy(pid_hbm.at[st, j, sl], pidxb[b], isem[b])

    def wait_idx(b):
        pltpu.make_async_copy(seq_hbm.at[0, 0, pl.ds(0, UNIT_ROWS)],
                              idxb[b], isem[b]).wait()
        pltpu.make_async_copy(pid_hbm.at[0, 0, pl.ds(0, UNIT_ROWS)],
                              pidxb[b], isem[b]).wait()

    def start_pos(b):
        pltpu.async_copy(pos_hbm.at[pidxb[b]], trows[b], psem[b])

    def wait_pos(b):
        pltpu.make_async_copy(pos_hbm.at[pidxb[b]], trows[b], psem[b]).wait()

    def out_copies(u, start):
        st, j, half = _coords(u)
        for sp in range(SI_PER_UNIT):
            for ht in range(8):
                cp = pltpu.make_async_copy(
                    hbuf.at[sp, pl.ds(ht * 1024, 1024)],
                    out_hbm.at[st * 8 + SI_PER_UNIT * half + sp, ht, j],
                    osem)
                if start:
                    cp.start()
                else:
                    cp.wait()

    stage(0, u0)
    stage(1, u0 + 1)
    wait_idx(0)
    start_pos(0)

    def group_body(g, carry):
        for b in range(NBUF):
            u = u0 + g * NBUF + b
            nb = (b + 1) % NBUF

            wait_pos(b)
            tok_cp = pltpu.async_copy(tok_hbm.at[idxb[b]], trows[b],
                                      tsem[b], add=True)

            @pl.when(u + 1 < u_end)
            def _():
                wait_idx(nb)
                start_pos(nb)

            tok_cp.wait()

            @pl.when(u > u0)
            def _():
                out_copies(u, start=False)

            def row_body(r, c):
                sp = r // 128
                bi = r - sp * 128
                spv = jnp.full((LANES,), sp, jnp.int32)
                for hb in range(HIDDEN // LANES):
                    v = trows[b][r, pl.ds(16 * hb, LANES)]
                    idxv = hb_const[hb] + bi
                    plsc.store_scatter(hbuf, [spv, idxv], v)
                return c

            lax.fori_loop(0, UNIT_ROWS, row_body, 0, unroll=False)

            out_copies(u, start=True)

            @pl.when(u + 2 < u_end)
            def _():
                stage(b, u + 2)
        return carry

    lax.fori_loop(0, UPW // NBUF, group_body, 0, unroll=False)
    out_copies(u_end - 1, start=False)


def kernel(sequence, position_ids, token_table, pos_table):
    seq_r = (sequence.T.reshape(ST, 8, BT, 128).transpose(0, 2, 1, 3)
             .reshape(ST, BT, 1024).astype(jnp.int32))
    pid_r = (position_ids.T.reshape(ST, 8, BT, 128).transpose(0, 2, 1, 3)
             .reshape(ST, BT, 1024).astype(jnp.int32))

    mesh = plsc.VectorSubcoreMesh(core_axis_name="c", subcore_axis_name="s",
                                  num_cores=NUM_CORES, num_subcores=NUM_SUBCORES)
    scratch = (
        [pltpu.VMEM((UNIT_ROWS,), jnp.int32) for _ in range(2 * NBUF)]
        + [pltpu.VMEM((UNIT_ROWS, HIDDEN), jnp.float32) for _ in range(NBUF)]
        + [pltpu.VMEM((SI_PER_UNIT, 8 * 1024), jnp.float32)]
        + [pltpu.SemaphoreType.DMA for _ in range(3 * NBUF + 1)]
    )
    emb = functools.partial(
        pl.kernel,
        out_type=jax.ShapeDtypeStruct((SEQ, 8, BT, 1024), jnp.float32),
        mesh=mesh,
        scratch_types=scratch,
        compiler_params=pltpu.CompilerParams(use_tc_tiling_on_sc=False,
                                             needs_layout_passes=False),
    )(_emb_body)

    x = emb(seq_r, pid_r, token_table, pos_table)
    return (x.reshape(SEQ, 8, BT, 8, 128).transpose(2, 4, 0, 1, 3)
            .reshape(BATCH, SEQ, HIDDEN))
